# baseline (speedup 1.0000x reference)
import functools

import jax
import jax.numpy as jnp
from jax import lax
from jax.experimental import pallas as pl
from jax.experimental.pallas import tpu as pltpu
from jax.experimental.pallas import tpu_sc as plsc

NC = 2
NS = 16
NW = NC * NS

NQ = 4
ND = 8
L = 16
BT = 128
NBUF = 2


def _embed_t(indices_t, table_t_flat, T, D, Bb):
    BQ = Bb // NQ
    NT = BQ // BT
    assert D == NW // NQ * ND

    mesh = plsc.VectorSubcoreMesh(core_axis_name="c", subcore_axis_name="s")

    @functools.partial(
        pl.kernel,
        mesh=mesh,
        out_type=jax.ShapeDtypeStruct((T, D // ND, Bb // BT, ND, BT),
                                      jnp.float32),
        scratch_types=[
            [pltpu.VMEM((BQ,), jnp.int32) for _ in range(NBUF)],
            [pltpu.VMEM((NT, ND, BT), jnp.float32) for _ in range(NBUF)],
            pltpu.VMEM((D * 128,), jnp.float32),
            [pltpu.SemaphoreType.DMA for _ in range(NBUF)],
            [pltpu.SemaphoreType.DMA for _ in range(NBUF)],
        ],
        compiler_params=pltpu.CompilerParams(
            use_tc_tiling_on_sc=False, needs_layout_passes=False
        ),
    )
    def k(idx_hbm, tab_hbm, out_hbm, idx_v, rows_v, tab_v, sem_idx, sem_out):
        wid = lax.axis_index("s") * NC + lax.axis_index("c")
        dt = wid // NQ
        q = wid % NQ
        b0 = q * BQ
        dbias = dt * (ND * 128)

        pltpu.sync_copy(tab_hbm, tab_v)

        def idx_copy(t, b):
            return pltpu.make_async_copy(
                idx_hbm.at[t, pl.ds(b0, BQ)], idx_v[b], sem_idx[b]
            )

        def out_copy(t, b):
            return pltpu.make_async_copy(
                rows_v[b],
                out_hbm.at[t, dt, pl.ds(q * NT, NT)],
                sem_out[b],
            )

        for b in range(NBUF):
            idx_copy(b, b).start()

        def body(g, _):
            for b in range(NBUF):
                t = g * NBUF + b
                idx_copy(t, b).wait()

                @pl.when(g > 0)
                def _reuse():
                    out_copy(t - NBUF, b).wait()

                @plsc.parallel_loop(0, NT, unroll=2)
                def gather(bt):
                    base = bt * BT
                    addr = [
                        idx_v[b][pl.ds(base + i * L, L)] + dbias
                        for i in range(BT // L)
                    ]
                    for j in range(ND):
                        for i in range(BT // L):
                            val = plsc.load_gather(tab_v, [addr[i]])
                            rows_v[b][bt, j, pl.ds(i * L, L)] = val
                            addr[i] = addr[i] + 128

                out_copy(t, b).start()
                tn = jnp.minimum(t + NBUF, T - 1)
                idx_copy(tn, b).start()
            return ()

        lax.fori_loop(0, T // NBUF, body, (), unroll=False)

        for b in range(NBUF):
            idx_copy(0, b).wait()
            out_copy(T - NBUF + b, b).wait()

    return k(indices_t, table_t_flat)


def kernel(indices, table):
    Bb, T = indices.shape
    V, D = table.shape
    indices_t = indices.T
    table_t = jnp.zeros((D, 128), table.dtype).at[:, :V].set(table.T)
    out5 = _embed_t(indices_t, table_t.reshape(-1), T, D, Bb)
    return out5.transpose(2, 4, 0, 1, 3).reshape(Bb, T, D)


# speedup vs baseline: 2.1222x; 2.1222x over previous
import functools

import jax
import jax.numpy as jnp
from jax import lax
from jax.experimental import pallas as pl
from jax.experimental.pallas import tpu as pltpu
from jax.experimental.pallas import tpu_sc as plsc

NC = 2
NS = 16
NW = NC * NS

NQ = 4
ND = 8
L = 16
BT = 128
NBUF = 2


def _embed_t(indices_t, table_t_flat, T, D, Bb):
    BQ = Bb // NQ
    NT = BQ // BT
    assert D == NW // NQ * ND

    mesh = plsc.VectorSubcoreMesh(core_axis_name="c", subcore_axis_name="s")

    @functools.partial(
        pl.kernel,
        mesh=mesh,
        out_type=jax.ShapeDtypeStruct((T, D // ND, Bb // BT, ND, BT),
                                      jnp.float32),
        scratch_types=[
            [pltpu.VMEM((BQ,), jnp.int32) for _ in range(NBUF)],
            [pltpu.VMEM((NT, ND, BT), jnp.float32) for _ in range(NBUF)],
            pltpu.VMEM((D * 128,), jnp.float32),
            [pltpu.SemaphoreType.DMA for _ in range(NBUF)],
            [pltpu.SemaphoreType.DMA for _ in range(NBUF)],
        ],
        compiler_params=pltpu.CompilerParams(
            use_tc_tiling_on_sc=False, needs_layout_passes=False
        ),
    )
    def k(idx_hbm, tab_hbm, out_hbm, idx_v, rows_v, tab_v, sem_idx, sem_out):
        wid = lax.axis_index("s") * NC + lax.axis_index("c")
        dt = wid // NQ
        q = wid % NQ
        b0 = q * BQ
        dbias = dt * (ND * 128)

        pltpu.sync_copy(tab_hbm, tab_v)

        def idx_copy(t, b):
            return pltpu.make_async_copy(
                idx_hbm.at[t, pl.ds(b0, BQ)], idx_v[b], sem_idx[b]
            )

        def out_copy(t, b):
            return pltpu.make_async_copy(
                rows_v[b],
                out_hbm.at[t, dt, pl.ds(q * NT, NT)],
                sem_out[b],
            )

        for b in range(NBUF):
            idx_copy(b, b).start()

        def body(g, _):
            for b in range(NBUF):
                t = g * NBUF + b
                idx_copy(t, b).wait()

                @pl.when(g > 0)
                def _reuse():
                    out_copy(t - NBUF, b).wait()

                @plsc.parallel_loop(0, NT)
                def gather(bt):
                    base = bt * BT
                    addr = [
                        idx_v[b][pl.ds(base + i * L, L)] + dbias
                        for i in range(BT // L)
                    ]
                    for j in range(ND):
                        for i in range(BT // L):
                            val = plsc.load_gather(tab_v, [addr[i]])
                            rows_v[b][bt, j, pl.ds(i * L, L)] = val
                            addr[i] = addr[i] + 128

                out_copy(t, b).start()
                tn = jnp.minimum(t + NBUF, T - 1)
                idx_copy(tn, b).start()
            return ()

        lax.fori_loop(0, T // NBUF, body, (), unroll=False)

        for b in range(NBUF):
            idx_copy(0, b).wait()
            out_copy(T - NBUF + b, b).wait()

    return k(indices_t, table_t_flat)


def kernel(indices, table):
    Bb, T = indices.shape
    V, D = table.shape
    indices_t = indices.T
    table_t = jnp.zeros((D, 128), table.dtype).at[:, :V].set(table.T)
    out5 = _embed_t(indices_t, table_t.reshape(-1), T, D, Bb)
    return out5.transpose(2, 4, 0, 1, 3).reshape(Bb, T, D)
